# Initial kernel scaffold; baseline (speedup 1.0000x reference)
import jax
import jax.numpy as jnp
from jax.experimental import pallas as pl


def kernel(Ax, Aedge_index, Abatch, Gx, Gedge_index, Gbatch, add_features, params):
    raise NotImplementedError("write your pallas kernel here")


# speedup vs baseline: 1.0000x; 1.0000x over previous
import jax
import jax.numpy as jnp
import numpy as np
from jax.experimental import pallas as pl

N_NODES = 10000
HID = 128


def _seg_max0(vals, seg, n):
    out = jax.ops.segment_max(vals, seg, num_segments=n)
    return jnp.where(jnp.isfinite(out), out, 0.0)


def _segment_softmax(alpha, seg, n):
    m = jax.ops.segment_max(alpha, seg, num_segments=n)
    m = jnp.where(jnp.isfinite(m), m, 0.0)
    e = jnp.exp(alpha - m[seg])
    s = jax.ops.segment_sum(e, seg, num_segments=n)
    return e / (s[seg] + 1e-16)


def _prelu(x, a):
    return jnp.where(x >= 0, x, a * x)


def _bn(x, g, b):
    mu = x.mean(0)
    var = x.var(0)
    return (x - mu) / jnp.sqrt(var + 1e-5) * g + b


def _gat(x, src, dst, p, n):
    h = x @ p['W_gat']
    a = (h * p['att_src']).sum(-1)[src] + (h * p['att_dst']).sum(-1)[dst]
    a = jax.nn.leaky_relu(a, 0.2)
    alpha = _segment_softmax(a, dst, n)
    out = _seg_max0(alpha[:, None] * h[src], dst, n)
    return out + p['b_gat']


def _sage(x, src, dst, p, n):
    agg = _seg_max0(x[src], dst, n)
    return agg @ p['W_l'] + p['b_l'] + x @ p['W_r']


def _sag_pool_max(x, src, dst, p, n):
    agg = jax.ops.segment_sum(x[src], dst, num_segments=n)
    score = (agg @ p['W_rel'] + x @ p['W_root'] + p['b_pool']).reshape(-1)
    k = int(np.ceil(0.5 * n))
    top_scores, perm = jax.lax.top_k(score, k)
    return (x[perm] * jnp.tanh(top_scores)[:, None]).max(axis=0)


def _branch_max(x, edge_index, p, n):
    src, dst = edge_index[0], edge_index[1]
    h = _gat(x, src, dst, p, n)
    h = _prelu(_bn(h, p['g1'], p['be1']), p['a1'])
    h = _sage(h, src, dst, p, n)
    h = _prelu(_bn(h, p['g2'], p['be2']), p['a2'])
    return _sag_pool_max(h, src, dst, p, n)


def _head_body(pool_ref, W1, b1, a1, W21, b21, a2, W22, b22, W3, b3, Wm, bm,
               iv_ref, tg_ref):
    pool = pool_ref[...]
    x = pool @ W1[...] + b1[...]
    x = jnp.where(x >= 0, x, a1[0, 0] * x)
    h = x @ W21[...] + b21[...]
    h = jnp.where(h >= 0, h, a2[0, 0] * h)
    iv_ref[...] = h @ W22[...] + b22[...]
    tg_ref[...] = jnp.exp(x @ W3[...] + b3[...]) * jnp.tanh(x @ Wm[...] + bm[...])


def kernel(Ax, Aedge_index, Abatch, Gx, Gedge_index, Gbatch, add_features, params):
    Aemb = _branch_max(Ax, Aedge_index, params['A'], N_NODES)
    Gemb = _branch_max(Gx, Gedge_index, params['G'], N_NODES)
    pool = jnp.concatenate([Aemb, Gemb, add_features]).reshape(1, -1)
    hp = params['head']
    iv, tg = pl.pallas_call(
        _head_body,
        out_shape=(jax.ShapeDtypeStruct((1, 1), jnp.float32),
                   jax.ShapeDtypeStruct((1, 1), jnp.float32)),
    )(pool, hp['W1'], hp['b1'].reshape(1, -1), hp['a_fc1'].reshape(1, 1),
      hp['W21'], hp['b21'].reshape(1, -1), hp['a_fc2'].reshape(1, 1),
      hp['W22'], hp['b22'].reshape(1, -1), hp['W3'], hp['b3'].reshape(1, -1),
      hp['Wm'], hp['bm'].reshape(1, -1))
    return iv.reshape(1), tg.reshape(1)


# speedup vs baseline: 2.0438x; 2.0437x over previous
import functools

import jax
import jax.numpy as jnp
from jax import lax
from jax.experimental import pallas as pl
from jax.experimental.pallas import tpu as pltpu
from jax.experimental.pallas import tpu_sc as plsc

N = 10000
NP = 10240
E = 320000
D = 128
K_TOP = 5000

NC, NS = 2, 16
NW = NC * NS
OWN = NP // NW
FPW = D // NW
EPW = E // NW
CH = 6400
A2CH = 2000
NEG = -3.0e38

_mesh = plsc.VectorSubcoreMesh(core_axis_name="c", subcore_axis_name="s")


def _wid():
    return lax.axis_index("s") * NC + lax.axis_index("c")


def _lanes(ref, g):
    return ref[pl.ds(g * 16, 16)]


def _const16(v, dtype):
    return jnp.full((16,), v, dtype)


def _rmw_max(acc_ref, idx_list, val_list, act0):
    def cond(act):
        return jnp.any(act)

    def body(act):
        for idx, val in zip(idx_list, val_list):
            cur = plsc.load_gather(acc_ref, [idx], mask=act)
            plsc.store_scatter(acc_ref, [idx], jnp.maximum(cur, val), mask=act)
        done = jnp.ones((16,), jnp.bool_)
        for idx, val in zip(idx_list, val_list):
            cur2 = plsc.load_gather(acc_ref, [idx], mask=act)
            done = done & jnp.where(act, cur2 >= val, True)
        return act & (~done)

    lax.while_loop(cond, body, act0)


@functools.partial(
    pl.kernel,
    out_type=(jax.ShapeDtypeStruct((NP,), jnp.float32),
              jax.ShapeDtypeStruct((NP,), jnp.float32)),
    mesh=_mesh,
    compiler_params=pltpu.CompilerParams(needs_layout_passes=False),
    scratch_types=[
        pltpu.VMEM((NP,), jnp.float32),
        pltpu.VMEM((NP,), jnp.float32),
        pltpu.VMEM((OWN,), jnp.float32),
        pltpu.VMEM((OWN,), jnp.float32),
        pltpu.VMEM((CH,), jnp.int32),
        pltpu.VMEM((CH,), jnp.int32),
    ],
)
def _sc_a1(as_h, ad_h, src_h, dst_h, m_h, s_h, as_t, ad_t, m_t, s_t, src_c, dst_c):
    wid = _wid()
    lo = wid * OWN
    pltpu.sync_copy(as_h, as_t)
    pltpu.sync_copy(ad_h, ad_t)

    def init_b(i, _):
        m_t[pl.ds(i * 16, 16)] = _const16(NEG, jnp.float32)
        s_t[pl.ds(i * 16, 16)] = _const16(0.0, jnp.float32)
        return _
    lax.fori_loop(0, OWN // 16, init_b, None)

    def edge_logit(g):
        sv = _lanes(src_c, g)
        dv = _lanes(dst_c, g)
        a = plsc.load_gather(as_t, [sv]) + plsc.load_gather(ad_t, [dv])
        a = jnp.where(a >= 0, a, 0.2 * a)
        la = dv - _const16(0, jnp.int32) - jnp.full((16,), lo, jnp.int32)
        inr = (la >= 0) & (la < OWN)
        lac = jnp.clip(la, 0, OWN - 1)
        return a, lac, inr

    def phase1_chunk(c, _):
        pltpu.sync_copy(src_h.at[pl.ds(c * CH, CH)], src_c)
        pltpu.sync_copy(dst_h.at[pl.ds(c * CH, CH)], dst_c)

        def grp(g, _):
            a, lac, inr = edge_logit(g)
            _rmw_max(m_t, [lac], [a], inr)
            return _
        lax.fori_loop(0, CH // 16, grp, None)
        return _
    lax.fori_loop(0, E // CH, phase1_chunk, None)

    def phase2_chunk(c, _):
        pltpu.sync_copy(src_h.at[pl.ds(c * CH, CH)], src_c)
        pltpu.sync_copy(dst_h.at[pl.ds(c * CH, CH)], dst_c)

        def grp(g, _):
            a, lac, inr = edge_logit(g)
            mg = plsc.load_gather(m_t, [lac], mask=inr)
            e = jnp.exp(jnp.where(inr, a - mg, 0.0))
            plsc.addupdate_scatter(s_t, [lac], e, mask=inr)
            return _
        lax.fori_loop(0, CH // 16, grp, None)
        return _
    lax.fori_loop(0, E // CH, phase2_chunk, None)

    pltpu.sync_copy(m_t, m_h.at[pl.ds(lo, OWN)])
    pltpu.sync_copy(s_t, s_h.at[pl.ds(lo, OWN)])


@functools.partial(
    pl.kernel,
    out_type=jax.ShapeDtypeStruct((E,), jnp.float32),
    mesh=_mesh,
    compiler_params=pltpu.CompilerParams(needs_layout_passes=False),
    scratch_types=[
        pltpu.VMEM((NP,), jnp.float32),
        pltpu.VMEM((NP,), jnp.float32),
        pltpu.VMEM((NP,), jnp.float32),
        pltpu.VMEM((NP,), jnp.float32),
        pltpu.VMEM((A2CH,), jnp.int32),
        pltpu.VMEM((A2CH,), jnp.int32),
        pltpu.VMEM((A2CH,), jnp.float32),
    ],
)
def _sc_a2(as_h, ad_h, m_h, s_h, src_h, dst_h, al_h,
           as_t, ad_t, m_t, s_t, src_c, dst_c, al_c):
    wid = _wid()
    pltpu.sync_copy(as_h, as_t)
    pltpu.sync_copy(ad_h, ad_t)
    pltpu.sync_copy(m_h, m_t)
    pltpu.sync_copy(s_h, s_t)

    def chunk(c, _):
        base = wid * EPW + c * A2CH
        pltpu.sync_copy(src_h.at[pl.ds(base, A2CH)], src_c)
        pltpu.sync_copy(dst_h.at[pl.ds(base, A2CH)], dst_c)

        def grp(g, _):
            sv = _lanes(src_c, g)
            dv = _lanes(dst_c, g)
            a = plsc.load_gather(as_t, [sv]) + plsc.load_gather(ad_t, [dv])
            a = jnp.where(a >= 0, a, 0.2 * a)
            e = jnp.exp(a - plsc.load_gather(m_t, [dv]))
            al = e / (plsc.load_gather(s_t, [dv]) + 1e-16)
            al_c[pl.ds(g * 16, 16)] = al
            return _
        lax.fori_loop(0, A2CH // 16, grp, None)
        pltpu.sync_copy(al_c, al_h.at[pl.ds(base, A2CH)])
        return _
    lax.fori_loop(0, EPW // A2CH, chunk, None)


def _make_sc_b(mode):
    scratch = [
        pltpu.VMEM((NP * FPW,), jnp.float32),
        pltpu.VMEM((NP * FPW,), jnp.float32),
        pltpu.VMEM((CH,), jnp.int32),
        pltpu.VMEM((CH,), jnp.int32),
    ]
    if mode == 'max_w':
        scratch.append(pltpu.VMEM((CH,), jnp.float32))

    @functools.partial(
        pl.kernel,
        out_type=jax.ShapeDtypeStruct((NW * NP * FPW,), jnp.float32),
        mesh=_mesh,
        compiler_params=pltpu.CompilerParams(needs_layout_passes=False),
        scratch_types=scratch,
    )
    def body(*refs):
        if mode == 'max_w':
            tab_h, src_h, dst_h, al_h, out_h, tab_t, acc_t, src_c, dst_c, al_c = refs
        else:
            tab_h, src_h, dst_h, out_h, tab_t, acc_t, src_c, dst_c = refs
            al_c = None
        wid = _wid()
        base = wid * (NP * FPW)
        pltpu.sync_copy(tab_h.at[pl.ds(base, NP * FPW)], tab_t)

        init = 0.0 if mode == 'sum' else NEG

        def init_b(i, _):
            acc_t[pl.ds(i * 16, 16)] = _const16(init, jnp.float32)
            return _
        lax.fori_loop(0, NP * FPW // 16, init_b, None)

        def chunk(c, _):
            pltpu.sync_copy(src_h.at[pl.ds(c * CH, CH)], src_c)
            pltpu.sync_copy(dst_h.at[pl.ds(c * CH, CH)], dst_c)
            if al_c is not None:
                pltpu.sync_copy(al_h.at[pl.ds(c * CH, CH)], al_c)

            def grp(g, _):
                sv = _lanes(src_c, g) * FPW
                dv = _lanes(dst_c, g) * FPW
                vals, idxs = [], []
                for f in range(FPW):
                    v = plsc.load_gather(tab_t, [sv + f])
                    if al_c is not None:
                        v = v * _lanes(al_c, g)
                    vals.append(v)
                    idxs.append(dv + f)
                if mode == 'sum':
                    for idx, v in zip(idxs, vals):
                        plsc.addupdate_scatter(acc_t, [idx], v)
                else:
                    _rmw_max(acc_t, idxs, vals, jnp.ones((16,), jnp.bool_))
                return _
            lax.fori_loop(0, CH // 16, grp, None)
            return _
        lax.fori_loop(0, E // CH, chunk, None)

        pltpu.sync_copy(acc_t, out_h.at[pl.ds(base, NP * FPW)])

    return body


_sc_b_maxw = _make_sc_b('max_w')
_sc_b_max = _make_sc_b('max')
_sc_b_sum = _make_sc_b('sum')


def _to_cols(x):
    return x.reshape(NP, NW, FPW).transpose(1, 0, 2).reshape(-1)


def _from_cols(flat):
    return flat.reshape(NW, NP, FPW).transpose(1, 0, 2).reshape(NP, D)


BLK = 256
GRID = NP // BLK


def _t1_body(x_ref, w_ref, aw_ref, dw_ref, h_ref, as_ref, ad_ref):
    h = jnp.dot(x_ref[...], w_ref[...], preferred_element_type=jnp.float32)
    h_ref[...] = h
    as_ref[...] = jnp.sum(h * aw_ref[...], axis=1, keepdims=True)
    ad_ref[...] = jnp.sum(h * dw_ref[...], axis=1, keepdims=True)


_t1 = pl.pallas_call(
    _t1_body,
    grid=(GRID,),
    in_specs=[
        pl.BlockSpec((BLK, D), lambda i: (i, 0)),
        pl.BlockSpec((D, D), lambda i: (0, 0)),
        pl.BlockSpec((1, D), lambda i: (0, 0)),
        pl.BlockSpec((1, D), lambda i: (0, 0)),
    ],
    out_specs=[
        pl.BlockSpec((BLK, D), lambda i: (i, 0)),
        pl.BlockSpec((BLK, 1), lambda i: (i, 0)),
        pl.BlockSpec((BLK, 1), lambda i: (i, 0)),
    ],
    out_shape=[
        jax.ShapeDtypeStruct((NP, D), jnp.float32),
        jax.ShapeDtypeStruct((NP, 1), jnp.float32),
        jax.ShapeDtypeStruct((NP, 1), jnp.float32),
    ],
)


def _stats(u, i, ss_ref):
    gid = i * BLK + lax.broadcasted_iota(jnp.int32, (BLK, D), 0)
    um = jnp.where(gid < N, u, 0.0)

    @pl.when(i == 0)
    def _():
        ss_ref[...] = jnp.zeros((8, D), jnp.float32)

    ss_ref[0:1, :] += jnp.sum(um, axis=0, keepdims=True)
    ss_ref[1:2, :] += jnp.sum(um * um, axis=0, keepdims=True)


def _t2a_body(acc_ref, bg_ref, u_ref, ss_ref):
    i = pl.program_id(0)
    a = acc_ref[...]
    u = jnp.where(a < -1e38, 0.0, a) + bg_ref[...]
    u_ref[...] = u
    _stats(u, i, ss_ref)


_t2a = pl.pallas_call(
    _t2a_body,
    grid=(GRID,),
    in_specs=[
        pl.BlockSpec((BLK, D), lambda i: (i, 0)),
        pl.BlockSpec((1, D), lambda i: (0, 0)),
    ],
    out_specs=[
        pl.BlockSpec((BLK, D), lambda i: (i, 0)),
        pl.BlockSpec((8, D), lambda i: (0, 0)),
    ],
    out_shape=[
        jax.ShapeDtypeStruct((NP, D), jnp.float32),
        jax.ShapeDtypeStruct((8, D), jnp.float32),
    ],
)


def _t3a_body(agg_ref, h1_ref, wl_ref, wr_ref, bl_ref, y_ref, ss_ref):
    i = pl.program_id(0)
    ag = agg_ref[...]
    ag = jnp.where(ag < -1e38, 0.0, ag)
    y = (jnp.dot(ag, wl_ref[...], preferred_element_type=jnp.float32)
         + jnp.dot(h1_ref[...], wr_ref[...], preferred_element_type=jnp.float32)
         + bl_ref[...])
    y_ref[...] = y
    _stats(y, i, ss_ref)


_t3a = pl.pallas_call(
    _t3a_body,
    grid=(GRID,),
    in_specs=[
        pl.BlockSpec((BLK, D), lambda i: (i, 0)),
        pl.BlockSpec((BLK, D), lambda i: (i, 0)),
        pl.BlockSpec((D, D), lambda i: (0, 0)),
        pl.BlockSpec((D, D), lambda i: (0, 0)),
        pl.BlockSpec((1, D), lambda i: (0, 0)),
    ],
    out_specs=[
        pl.BlockSpec((BLK, D), lambda i: (i, 0)),
        pl.BlockSpec((8, D), lambda i: (0, 0)),
    ],
    out_shape=[
        jax.ShapeDtypeStruct((NP, D), jnp.float32),
        jax.ShapeDtypeStruct((8, D), jnp.float32),
    ],
)


def _t2b_body(u_ref, ss_ref, g_ref, b_ref, a_ref, out_ref):
    mu = ss_ref[0:1, :] / float(N)
    ex2 = ss_ref[1:2, :] / float(N)
    var = jnp.maximum(ex2 - mu * mu, 0.0)
    rstd = lax.rsqrt(var + 1e-5)
    xn = (u_ref[...] - mu) * rstd * g_ref[...] + b_ref[...]
    al = a_ref[0, 0]
    out_ref[...] = jnp.where(xn >= 0, xn, al * xn)


_t2b = pl.pallas_call(
    _t2b_body,
    grid=(GRID,),
    in_specs=[
        pl.BlockSpec((BLK, D), lambda i: (i, 0)),
        pl.BlockSpec((8, D), lambda i: (0, 0)),
        pl.BlockSpec((1, D), lambda i: (0, 0)),
        pl.BlockSpec((1, D), lambda i: (0, 0)),
        pl.BlockSpec((1, 1), lambda i: (0, 0)),
    ],
    out_specs=pl.BlockSpec((BLK, D), lambda i: (i, 0)),
    out_shape=jax.ShapeDtypeStruct((NP, D), jnp.float32),
)


def _t5_body(aggs_ref, h2_ref, wrel_ref, wroot_ref, bp_ref, emb_ref):
    score = (jnp.dot(aggs_ref[...], wrel_ref[...], preferred_element_type=jnp.float32)
             + jnp.dot(h2_ref[...], wroot_ref[...], preferred_element_type=jnp.float32)
             + bp_ref[0, 0])
    riota = lax.broadcasted_iota(jnp.int32, (NP, 1), 0)
    score = jnp.where(riota < N, score, NEG)
    bits = lax.bitcast_convert_type(score, jnp.uint32)
    neg = bits >= jnp.uint32(0x80000000)
    key = jnp.where(neg, ~bits, bits | jnp.uint32(0x80000000))

    def vbit(it, ans):
        cand = ans | (jnp.uint32(1) << jnp.uint32(31 - it))
        cnt = jnp.sum((key >= cand).astype(jnp.int32))
        return jnp.where(cnt >= K_TOP, cand, ans)

    ans = lax.fori_loop(0, 32, vbit, jnp.uint32(0))
    c_gt = jnp.sum((key > ans).astype(jnp.int32))
    need = K_TOP - c_gt
    eq = key == ans

    def ibit(it, ans2):
        cand = ans2 | (1 << (13 - it))
        cnt = jnp.sum((eq & (riota < cand)).astype(jnp.int32))
        return jnp.where(cnt <= need, cand, ans2)

    ans2 = lax.fori_loop(0, 14, ibit, 0)
    sel = (key > ans) | (eq & (riota < ans2))
    val = h2_ref[...] * jnp.tanh(score)
    emb_ref[...] = jnp.max(jnp.where(sel, val, NEG), axis=0, keepdims=True)


_t5 = pl.pallas_call(
    _t5_body,
    out_shape=jax.ShapeDtypeStruct((1, D), jnp.float32),
)


def _head_body(pool_ref, W1, b1, a1, W21, b21, a2, W22, b22, W3, b3, Wm, bm,
               iv_ref, tg_ref):
    pool = pool_ref[...]
    x = jnp.dot(pool, W1[...], preferred_element_type=jnp.float32) + b1[...]
    x = jnp.where(x >= 0, x, a1[0, 0] * x)
    h = jnp.dot(x, W21[...], preferred_element_type=jnp.float32) + b21[...]
    h = jnp.where(h >= 0, h, a2[0, 0] * h)
    iv_ref[...] = jnp.dot(h, W22[...], preferred_element_type=jnp.float32) + b22[...]
    tg_ref[...] = (jnp.exp(jnp.dot(x, W3[...], preferred_element_type=jnp.float32) + b3[...])
                   * jnp.tanh(jnp.dot(x, Wm[...], preferred_element_type=jnp.float32) + bm[...]))


_head = pl.pallas_call(
    _head_body,
    out_shape=(jax.ShapeDtypeStruct((1, 1), jnp.float32),
               jax.ShapeDtypeStruct((1, 1), jnp.float32)),
)


def _branch(x, ei, p):
    src = ei[0]
    dst = ei[1]
    x_p = jnp.pad(x, ((0, NP - N), (0, 0)))

    h, as2, ad2 = _t1(x_p, p['W_gat'], p['att_src'].reshape(1, D),
                      p['att_dst'].reshape(1, D))
    asv = as2.reshape(NP)
    adv = ad2.reshape(NP)

    m, s = _sc_a1(asv, adv, src, dst)
    alpha = _sc_a2(asv, adv, m, s, src, dst)

    gat_acc = _from_cols(_sc_b_maxw(_to_cols(h), src, dst, alpha))
    u, ss = _t2a(gat_acc, p['b_gat'].reshape(1, D))
    h1 = _t2b(u, ss, p['g1'].reshape(1, D), p['be1'].reshape(1, D),
              p['a1'].reshape(1, 1))

    sage_acc = _from_cols(_sc_b_max(_to_cols(h1), src, dst))
    y, ss2 = _t3a(sage_acc, h1, p['W_l'], p['W_r'], p['b_l'].reshape(1, D))
    h2 = _t2b(y, ss2, p['g2'].reshape(1, D), p['be2'].reshape(1, D),
              p['a2'].reshape(1, 1))

    aggsum = _from_cols(_sc_b_sum(_to_cols(h2), src, dst))
    emb = _t5(aggsum, h2, p['W_rel'], p['W_root'], p['b_pool'].reshape(1, 1))
    return emb


def kernel(Ax, Aedge_index, Abatch, Gx, Gedge_index, Gbatch, add_features, params):
    embA = _branch(Ax, Aedge_index, params['A'])
    embG = _branch(Gx, Gedge_index, params['G'])
    pool = jnp.concatenate([embA[0], embG[0], add_features]).reshape(1, -1)
    hp = params['head']
    iv, tg = _head(
        pool, hp['W1'], hp['b1'].reshape(1, -1), hp['a_fc1'].reshape(1, 1),
        hp['W21'], hp['b21'].reshape(1, -1), hp['a_fc2'].reshape(1, 1),
        hp['W22'], hp['b22'].reshape(1, -1), hp['W3'], hp['b3'].reshape(1, -1),
        hp['Wm'], hp['bm'].reshape(1, -1))
    return iv.reshape(1), tg.reshape(1)


# speedup vs baseline: 2.6405x; 1.2920x over previous
import functools

import jax
import jax.numpy as jnp
from jax import lax
from jax.experimental import pallas as pl
from jax.experimental.pallas import tpu as pltpu
from jax.experimental.pallas import tpu_sc as plsc

N = 10000
NP = 10240
E = 320000
D = 128
K_TOP = 5000

NC, NS = 2, 16
NW = NC * NS
OWN = NP // NW
FPW = D // NW
EPW = E // NW
CH = 6400
A2CH = 2000
NEG = -3.0e38

_mesh = plsc.VectorSubcoreMesh(core_axis_name="c", subcore_axis_name="s")


def _wid():
    return lax.axis_index("s") * NC + lax.axis_index("c")


def _lanes(ref, g):
    return ref[pl.ds(g * 16, 16)]


def _const16(v, dtype):
    return jnp.full((16,), v, dtype)


def _rmw_max_slow(acc_ref, idx_list, val_list, act0):
    def cond(act):
        return jnp.any(act)

    def body(act):
        for idx, val in zip(idx_list, val_list):
            cur = plsc.load_gather(acc_ref, [idx], mask=act)
            plsc.store_scatter(acc_ref, [idx], jnp.maximum(cur, val), mask=act)
        done = jnp.ones((16,), jnp.bool_)
        for idx, val in zip(idx_list, val_list):
            cur2 = plsc.load_gather(acc_ref, [idx], mask=act)
            done = done & jnp.where(act, cur2 >= val, True)
        return act & (~done)

    lax.while_loop(cond, body, act0)


def _rmw_max(acc_ref, tmp_ref, key, idx_list, val_list, act0):
    lid = lax.iota(jnp.int32, 16)
    plsc.store_scatter(tmp_ref, [key], lid, mask=act0)
    got = plsc.load_gather(tmp_ref, [key], mask=act0)
    win = act0 & (got == lid)
    for idx, val in zip(idx_list, val_list):
        cur = plsc.load_gather(acc_ref, [idx], mask=win)
        plsc.store_scatter(acc_ref, [idx], jnp.maximum(cur, val), mask=win)
    losers = act0 & (~win)

    @pl.when(jnp.any(losers))
    def _():
        _rmw_max_slow(acc_ref, idx_list, val_list, losers)


@functools.partial(
    pl.kernel,
    out_type=(jax.ShapeDtypeStruct((NP,), jnp.float32),
              jax.ShapeDtypeStruct((NP,), jnp.float32)),
    mesh=_mesh,
    compiler_params=pltpu.CompilerParams(needs_layout_passes=False),
    scratch_types=[
        pltpu.VMEM((NP,), jnp.float32),
        pltpu.VMEM((NP,), jnp.float32),
        pltpu.VMEM((OWN,), jnp.float32),
        pltpu.VMEM((OWN,), jnp.float32),
        pltpu.VMEM((OWN,), jnp.int32),
        pltpu.VMEM((CH,), jnp.int32),
        pltpu.VMEM((CH,), jnp.int32),
    ],
)
def _sc_a1(as_h, ad_h, src_h, dst_h, m_h, s_h, as_t, ad_t, m_t, s_t, tmp_t,
           src_c, dst_c):
    wid = _wid()
    lo = wid * OWN
    lov = jnp.full((16,), lo, jnp.int32)
    pltpu.sync_copy(as_h, as_t)
    pltpu.sync_copy(ad_h, ad_t)

    def init_b(i, _):
        m_t[pl.ds(i * 16, 16)] = _const16(NEG, jnp.float32)
        s_t[pl.ds(i * 16, 16)] = _const16(0.0, jnp.float32)
        return _
    lax.fori_loop(0, OWN // 16, init_b, None)

    def edge_logit(sv, dv, inr):
        a = plsc.load_gather(as_t, [sv]) + plsc.load_gather(ad_t, [dv])
        a = jnp.where(a >= 0, a, 0.2 * a)
        return a

    def phase1_chunk(c, _):
        pltpu.sync_copy(src_h.at[pl.ds(c * CH, CH)], src_c)
        pltpu.sync_copy(dst_h.at[pl.ds(c * CH, CH)], dst_c)

        def grp(g, _):
            dv = _lanes(dst_c, g)
            la = dv - lov
            inr = (la >= 0) & (la < OWN)

            @pl.when(jnp.any(inr))
            def _():
                sv = _lanes(src_c, g)
                a = edge_logit(sv, dv, inr)
                lac = jnp.clip(la, 0, OWN - 1)
                _rmw_max(m_t, tmp_t, lac, [lac], [a], inr)
            return _
        lax.fori_loop(0, CH // 16, grp, None)
        return _
    lax.fori_loop(0, E // CH, phase1_chunk, None)

    def phase2_chunk(c, _):
        pltpu.sync_copy(src_h.at[pl.ds(c * CH, CH)], src_c)
        pltpu.sync_copy(dst_h.at[pl.ds(c * CH, CH)], dst_c)

        @plsc.parallel_loop(0, CH // 16)
        def grp(g):
            dv = _lanes(dst_c, g)
            la = dv - lov
            inr = (la >= 0) & (la < OWN)
            sv = _lanes(src_c, g)
            a = edge_logit(sv, dv, inr)
            lac = jnp.clip(la, 0, OWN - 1)
            mg = plsc.load_gather(m_t, [lac], mask=inr)
            e = jnp.exp(jnp.where(inr, a - mg, 0.0))
            plsc.addupdate_scatter(s_t, [lac], e, mask=inr)
        return _
    lax.fori_loop(0, E // CH, phase2_chunk, None)

    pltpu.sync_copy(m_t, m_h.at[pl.ds(lo, OWN)])
    pltpu.sync_copy(s_t, s_h.at[pl.ds(lo, OWN)])


@functools.partial(
    pl.kernel,
    out_type=jax.ShapeDtypeStruct((E,), jnp.float32),
    mesh=_mesh,
    compiler_params=pltpu.CompilerParams(needs_layout_passes=False),
    scratch_types=[
        pltpu.VMEM((NP,), jnp.float32),
        pltpu.VMEM((NP,), jnp.float32),
        pltpu.VMEM((NP,), jnp.float32),
        pltpu.VMEM((NP,), jnp.float32),
        pltpu.VMEM((A2CH,), jnp.int32),
        pltpu.VMEM((A2CH,), jnp.int32),
        pltpu.VMEM((A2CH,), jnp.float32),
    ],
)
def _sc_a2(as_h, ad_h, m_h, s_h, src_h, dst_h, al_h,
           as_t, ad_t, m_t, s_t, src_c, dst_c, al_c):
    wid = _wid()
    pltpu.sync_copy(as_h, as_t)
    pltpu.sync_copy(ad_h, ad_t)
    pltpu.sync_copy(m_h, m_t)
    pltpu.sync_copy(s_h, s_t)

    def chunk(c, _):
        base = wid * EPW + c * A2CH
        pltpu.sync_copy(src_h.at[pl.ds(base, A2CH)], src_c)
        pltpu.sync_copy(dst_h.at[pl.ds(base, A2CH)], dst_c)

        @plsc.parallel_loop(0, A2CH // 16)
        def grp(g):
            sv = _lanes(src_c, g)
            dv = _lanes(dst_c, g)
            a = plsc.load_gather(as_t, [sv]) + plsc.load_gather(ad_t, [dv])
            a = jnp.where(a >= 0, a, 0.2 * a)
            e = jnp.exp(a - plsc.load_gather(m_t, [dv]))
            al = e / (plsc.load_gather(s_t, [dv]) + 1e-16)
            al_c[pl.ds(g * 16, 16)] = al
        pltpu.sync_copy(al_c, al_h.at[pl.ds(base, A2CH)])
        return _
    lax.fori_loop(0, EPW // A2CH, chunk, None)


def _make_sc_b(mode):
    scratch = [
        pltpu.VMEM((NP * FPW,), jnp.float32),
        pltpu.VMEM((NP * FPW,), jnp.float32),
        pltpu.VMEM((NP,), jnp.int32),
        pltpu.VMEM((CH,), jnp.int32),
        pltpu.VMEM((CH,), jnp.int32),
    ]
    if mode == 'max_w':
        scratch.append(pltpu.VMEM((CH,), jnp.float32))

    @functools.partial(
        pl.kernel,
        out_type=jax.ShapeDtypeStruct((NW * NP * FPW,), jnp.float32),
        mesh=_mesh,
        compiler_params=pltpu.CompilerParams(needs_layout_passes=False),
        scratch_types=scratch,
    )
    def body(*refs):
        if mode == 'max_w':
            tab_h, src_h, dst_h, al_h, out_h, tab_t, acc_t, tmp_t, src_c, dst_c, al_c = refs
        else:
            tab_h, src_h, dst_h, out_h, tab_t, acc_t, tmp_t, src_c, dst_c = refs
            al_c = None
        wid = _wid()
        base = wid * (NP * FPW)
        pltpu.sync_copy(tab_h.at[pl.ds(base, NP * FPW)], tab_t)

        init = 0.0 if mode == 'sum' else NEG

        @plsc.parallel_loop(0, NP * FPW // 16)
        def init_b(i):
            acc_t[pl.ds(i * 16, 16)] = _const16(init, jnp.float32)

        def gather_vals(g):
            sv = _lanes(src_c, g) * FPW
            dv0 = _lanes(dst_c, g)
            dv = dv0 * FPW
            vals, idxs = [], []
            for f in range(FPW):
                v = plsc.load_gather(tab_t, [sv + f])
                if al_c is not None:
                    v = v * _lanes(al_c, g)
                vals.append(v)
                idxs.append(dv + f)
            return dv0, idxs, vals

        def chunk(c, _):
            pltpu.sync_copy(src_h.at[pl.ds(c * CH, CH)], src_c)
            pltpu.sync_copy(dst_h.at[pl.ds(c * CH, CH)], dst_c)
            if al_c is not None:
                pltpu.sync_copy(al_h.at[pl.ds(c * CH, CH)], al_c)

            if mode == 'sum':
                @plsc.parallel_loop(0, CH // 16)
                def grp(g):
                    _, idxs, vals = gather_vals(g)
                    for idx, v in zip(idxs, vals):
                        plsc.addupdate_scatter(acc_t, [idx], v)
            else:
                def grp(g, _):
                    dv0, idxs, vals = gather_vals(g)
                    _rmw_max(acc_t, tmp_t, dv0, idxs, vals,
                             jnp.ones((16,), jnp.bool_))
                    return _
                lax.fori_loop(0, CH // 16, grp, None)
            return _
        lax.fori_loop(0, E // CH, chunk, None)

        pltpu.sync_copy(acc_t, out_h.at[pl.ds(base, NP * FPW)])

    return body


_sc_b_maxw = _make_sc_b('max_w')
_sc_b_max = _make_sc_b('max')
_sc_b_sum = _make_sc_b('sum')


def _to_cols(x):
    return x.reshape(NP, NW, FPW).transpose(1, 0, 2).reshape(-1)


def _from_cols(flat):
    return flat.reshape(NW, NP, FPW).transpose(1, 0, 2).reshape(NP, D)


BLK = 256
GRID = NP // BLK


def _t1_body(x_ref, w_ref, aw_ref, dw_ref, h_ref, as_ref, ad_ref):
    h = jnp.dot(x_ref[...], w_ref[...], preferred_element_type=jnp.float32)
    h_ref[...] = h
    as_ref[...] = jnp.sum(h * aw_ref[...], axis=1, keepdims=True)
    ad_ref[...] = jnp.sum(h * dw_ref[...], axis=1, keepdims=True)


_t1 = pl.pallas_call(
    _t1_body,
    grid=(GRID,),
    in_specs=[
        pl.BlockSpec((BLK, D), lambda i: (i, 0)),
        pl.BlockSpec((D, D), lambda i: (0, 0)),
        pl.BlockSpec((1, D), lambda i: (0, 0)),
        pl.BlockSpec((1, D), lambda i: (0, 0)),
    ],
    out_specs=[
        pl.BlockSpec((BLK, D), lambda i: (i, 0)),
        pl.BlockSpec((BLK, 1), lambda i: (i, 0)),
        pl.BlockSpec((BLK, 1), lambda i: (i, 0)),
    ],
    out_shape=[
        jax.ShapeDtypeStruct((NP, D), jnp.float32),
        jax.ShapeDtypeStruct((NP, 1), jnp.float32),
        jax.ShapeDtypeStruct((NP, 1), jnp.float32),
    ],
)


def _stats(u, i, ss_ref):
    gid = i * BLK + lax.broadcasted_iota(jnp.int32, (BLK, D), 0)
    um = jnp.where(gid < N, u, 0.0)

    @pl.when(i == 0)
    def _():
        ss_ref[...] = jnp.zeros((8, D), jnp.float32)

    ss_ref[0:1, :] += jnp.sum(um, axis=0, keepdims=True)
    ss_ref[1:2, :] += jnp.sum(um * um, axis=0, keepdims=True)


def _t2a_body(acc_ref, bg_ref, u_ref, ss_ref):
    i = pl.program_id(0)
    a = acc_ref[...]
    u = jnp.where(a < -1e38, 0.0, a) + bg_ref[...]
    u_ref[...] = u
    _stats(u, i, ss_ref)


_t2a = pl.pallas_call(
    _t2a_body,
    grid=(GRID,),
    in_specs=[
        pl.BlockSpec((BLK, D), lambda i: (i, 0)),
        pl.BlockSpec((1, D), lambda i: (0, 0)),
    ],
    out_specs=[
        pl.BlockSpec((BLK, D), lambda i: (i, 0)),
        pl.BlockSpec((8, D), lambda i: (0, 0)),
    ],
    out_shape=[
        jax.ShapeDtypeStruct((NP, D), jnp.float32),
        jax.ShapeDtypeStruct((8, D), jnp.float32),
    ],
)


def _t3a_body(agg_ref, h1_ref, wl_ref, wr_ref, bl_ref, y_ref, ss_ref):
    i = pl.program_id(0)
    ag = agg_ref[...]
    ag = jnp.where(ag < -1e38, 0.0, ag)
    y = (jnp.dot(ag, wl_ref[...], preferred_element_type=jnp.float32)
         + jnp.dot(h1_ref[...], wr_ref[...], preferred_element_type=jnp.float32)
         + bl_ref[...])
    y_ref[...] = y
    _stats(y, i, ss_ref)


_t3a = pl.pallas_call(
    _t3a_body,
    grid=(GRID,),
    in_specs=[
        pl.BlockSpec((BLK, D), lambda i: (i, 0)),
        pl.BlockSpec((BLK, D), lambda i: (i, 0)),
        pl.BlockSpec((D, D), lambda i: (0, 0)),
        pl.BlockSpec((D, D), lambda i: (0, 0)),
        pl.BlockSpec((1, D), lambda i: (0, 0)),
    ],
    out_specs=[
        pl.BlockSpec((BLK, D), lambda i: (i, 0)),
        pl.BlockSpec((8, D), lambda i: (0, 0)),
    ],
    out_shape=[
        jax.ShapeDtypeStruct((NP, D), jnp.float32),
        jax.ShapeDtypeStruct((8, D), jnp.float32),
    ],
)


def _t2b_body(u_ref, ss_ref, g_ref, b_ref, a_ref, out_ref):
    mu = ss_ref[0:1, :] / float(N)
    ex2 = ss_ref[1:2, :] / float(N)
    var = jnp.maximum(ex2 - mu * mu, 0.0)
    rstd = lax.rsqrt(var + 1e-5)
    xn = (u_ref[...] - mu) * rstd * g_ref[...] + b_ref[...]
    al = a_ref[0, 0]
    out_ref[...] = jnp.where(xn >= 0, xn, al * xn)


_t2b = pl.pallas_call(
    _t2b_body,
    grid=(GRID,),
    in_specs=[
        pl.BlockSpec((BLK, D), lambda i: (i, 0)),
        pl.BlockSpec((8, D), lambda i: (0, 0)),
        pl.BlockSpec((1, D), lambda i: (0, 0)),
        pl.BlockSpec((1, D), lambda i: (0, 0)),
        pl.BlockSpec((1, 1), lambda i: (0, 0)),
    ],
    out_specs=pl.BlockSpec((BLK, D), lambda i: (i, 0)),
    out_shape=jax.ShapeDtypeStruct((NP, D), jnp.float32),
)


def _t5_body(aggs_ref, h2_ref, wrel_ref, wroot_ref, bp_ref, emb_ref):
    score = (jnp.dot(aggs_ref[...], wrel_ref[...], preferred_element_type=jnp.float32)
             + jnp.dot(h2_ref[...], wroot_ref[...], preferred_element_type=jnp.float32)
             + bp_ref[0, 0])
    riota = lax.broadcasted_iota(jnp.int32, (NP, 1), 0)
    score = jnp.where(riota < N, score, NEG)
    bits = lax.bitcast_convert_type(score, jnp.uint32)
    neg = bits >= jnp.uint32(0x80000000)
    key = jnp.where(neg, ~bits, bits | jnp.uint32(0x80000000))

    def vbit(it, ans):
        cand = ans | (jnp.uint32(1) << jnp.uint32(31 - it))
        cnt = jnp.sum((key >= cand).astype(jnp.int32))
        return jnp.where(cnt >= K_TOP, cand, ans)

    ans = lax.fori_loop(0, 32, vbit, jnp.uint32(0))
    c_gt = jnp.sum((key > ans).astype(jnp.int32))
    need = K_TOP - c_gt
    eq = key == ans

    def ibit(it, ans2):
        cand = ans2 | (1 << (13 - it))
        cnt = jnp.sum((eq & (riota < cand)).astype(jnp.int32))
        return jnp.where(cnt <= need, cand, ans2)

    ans2 = lax.fori_loop(0, 14, ibit, 0)
    sel = (key > ans) | (eq & (riota < ans2))
    val = h2_ref[...] * jnp.tanh(score)
    emb_ref[...] = jnp.max(jnp.where(sel, val, NEG), axis=0, keepdims=True)


_t5 = pl.pallas_call(
    _t5_body,
    out_shape=jax.ShapeDtypeStruct((1, D), jnp.float32),
)


def _head_body(pool_ref, W1, b1, a1, W21, b21, a2, W22, b22, W3, b3, Wm, bm,
               iv_ref, tg_ref):
    pool = pool_ref[...]
    x = jnp.dot(pool, W1[...], preferred_element_type=jnp.float32) + b1[...]
    x = jnp.where(x >= 0, x, a1[0, 0] * x)
    h = jnp.dot(x, W21[...], preferred_element_type=jnp.float32) + b21[...]
    h = jnp.where(h >= 0, h, a2[0, 0] * h)
    iv_ref[...] = jnp.dot(h, W22[...], preferred_element_type=jnp.float32) + b22[...]
    tg_ref[...] = (jnp.exp(jnp.dot(x, W3[...], preferred_element_type=jnp.float32) + b3[...])
                   * jnp.tanh(jnp.dot(x, Wm[...], preferred_element_type=jnp.float32) + bm[...]))


_head = pl.pallas_call(
    _head_body,
    out_shape=(jax.ShapeDtypeStruct((1, 1), jnp.float32),
               jax.ShapeDtypeStruct((1, 1), jnp.float32)),
)


def _branch(x, ei, p):
    src = ei[0]
    dst = ei[1]
    x_p = jnp.pad(x, ((0, NP - N), (0, 0)))

    h, as2, ad2 = _t1(x_p, p['W_gat'], p['att_src'].reshape(1, D),
                      p['att_dst'].reshape(1, D))
    asv = as2.reshape(NP)
    adv = ad2.reshape(NP)

    m, s = _sc_a1(asv, adv, src, dst)
    alpha = _sc_a2(asv, adv, m, s, src, dst)

    gat_acc = _from_cols(_sc_b_maxw(_to_cols(h), src, dst, alpha))
    u, ss = _t2a(gat_acc, p['b_gat'].reshape(1, D))
    h1 = _t2b(u, ss, p['g1'].reshape(1, D), p['be1'].reshape(1, D),
              p['a1'].reshape(1, 1))

    sage_acc = _from_cols(_sc_b_max(_to_cols(h1), src, dst))
    y, ss2 = _t3a(sage_acc, h1, p['W_l'], p['W_r'], p['b_l'].reshape(1, D))
    h2 = _t2b(y, ss2, p['g2'].reshape(1, D), p['be2'].reshape(1, D),
              p['a2'].reshape(1, 1))

    aggsum = _from_cols(_sc_b_sum(_to_cols(h2), src, dst))
    emb = _t5(aggsum, h2, p['W_rel'], p['W_root'], p['b_pool'].reshape(1, 1))
    return emb


def kernel(Ax, Aedge_index, Abatch, Gx, Gedge_index, Gbatch, add_features, params):
    embA = _branch(Ax, Aedge_index, params['A'])
    embG = _branch(Gx, Gedge_index, params['G'])
    pool = jnp.concatenate([embA[0], embG[0], add_features]).reshape(1, -1)
    hp = params['head']
    iv, tg = _head(
        pool, hp['W1'], hp['b1'].reshape(1, -1), hp['a_fc1'].reshape(1, 1),
        hp['W21'], hp['b21'].reshape(1, -1), hp['a_fc2'].reshape(1, 1),
        hp['W22'], hp['b22'].reshape(1, -1), hp['W3'], hp['b3'].reshape(1, -1),
        hp['Wm'], hp['bm'].reshape(1, -1))
    return iv.reshape(1), tg.reshape(1)
